# Initial kernel scaffold; baseline (speedup 1.0000x reference)
import jax
import jax.numpy as jnp
from jax.experimental import pallas as pl


def kernel(x):
    raise NotImplementedError("write your pallas kernel here")


# speedup vs baseline: 19.6433x; 19.6433x over previous
import functools

import jax
import jax.numpy as jnp
from jax import lax
from jax.experimental import pallas as pl
from jax.experimental.pallas import tpu as pltpu
from jax.experimental.pallas import tpu_sc as plsc

NC = 2
NS = 16
L = 16
NW = NC * NS

TOPK_FRAC = 0.2
MASK31 = 0x7FFFFFFF

LEVELS = ((21, 1024), (11, 1024), (0, 2048))
HIST_WORDS = 2048 * L
SUF_WORDS = 2048


@functools.lru_cache(maxsize=None)
def _build(rows: int, row: int, k: int):
    assert rows % NW == 0 and row % L == 0
    rpw = rows // NW
    nv = row // L

    mesh = plsc.VectorSubcoreMesh(
        core_axis_name="c", subcore_axis_name="s", num_cores=NC, num_subcores=NS
    )

    @functools.partial(
        pl.kernel,
        out_type=jax.ShapeDtypeStruct((rows, row), jnp.float32),
        mesh=mesh,
        scratch_types=[
            pltpu.VMEM((row,), jnp.float32),
            pltpu.VMEM((HIST_WORDS,), jnp.int32),
            pltpu.VMEM((SUF_WORDS,), jnp.int32),
        ],
        compiler_params=pltpu.CompilerParams(needs_layout_passes=False),
    )
    def topk_rows(x_hbm, out_hbm, row_v, hist_v, suf_v):
        wid = lax.axis_index("s") * NC + lax.axis_index("c")
        lane = lax.iota(jnp.int32, L)
        ones = jnp.ones((L,), jnp.int32)
        zeros = jnp.zeros((L,), jnp.int32)

        def per_row(r, carry):
            rid = wid * rpw + r
            pltpu.sync_copy(x_hbm.at[rid], row_v)

            prefix = jnp.int32(0)
            cnt_above = jnp.int32(0)
            for li, (shift, width) in enumerate(LEVELS):
                def zbody(i, c):
                    hist_v[pl.ds(i * L, L)] = zeros
                    return c

                lax.fori_loop(0, width, zbody, 0)

                laneoff = lane * width
                digit_bits = width.bit_length() - 1
                mask_shift = shift + digit_bits

                def hbody(i, c, shift=shift, width=width, laneoff=laneoff,
                          mask_shift=mask_shift, prefix=prefix, first=(li == 0)):
                    v = row_v[pl.ds(i * L, L)]
                    b = lax.bitcast_convert_type(v, jnp.int32) & MASK31
                    d = (b >> shift) & (width - 1)
                    idx = laneoff + d
                    if first:
                        plsc.addupdate_scatter(hist_v, [idx], ones)
                    else:
                        m = (b >> mask_shift) == prefix
                        plsc.addupdate_scatter(hist_v, [idx], ones, mask=m)
                    return c

                lax.fori_loop(0, nv, hbody, 0)

                nb = width // L

                def cbody(j, car, width=width, nb=nb, cnt_above=cnt_above):
                    carry_sum, cntv = car
                    cblk = nb - 1 - j
                    acc = hist_v[pl.ds(cblk * L, L)]
                    for l in range(1, L):
                        acc = acc + hist_v[pl.ds(l * width + cblk * L, L)]
                    sfx = lax.rev(jnp.cumsum(lax.rev(acc, (0,))), (0,)) + carry_sum
                    suf_v[pl.ds(cblk * L, L)] = sfx
                    cntv = cntv + (cnt_above + sfx >= k).astype(jnp.int32)
                    return carry_sum + jnp.sum(acc), cntv

                _, cntv = lax.fori_loop(0, nb, cbody, (jnp.int32(0), zeros))
                dsel = jnp.sum(cntv) - 1

                def ubody(j, acc2, dsel=dsel):
                    dv = lax.iota(jnp.int32, L) + j * L
                    sv = suf_v[pl.ds(j * L, L)]
                    return acc2 + jnp.where(dv == dsel + 1, sv, 0)

                accv = lax.fori_loop(0, nb, ubody, zeros)
                cnt_above = cnt_above + jnp.sum(accv)
                prefix = prefix * width + dsel

            tbits = prefix

            def mbody(i, c):
                v = row_v[pl.ds(i * L, L)]
                b = lax.bitcast_convert_type(v, jnp.int32) & MASK31
                row_v[pl.ds(i * L, L)] = jnp.where(b >= tbits, v, jnp.float32(0.0))
                return c

            lax.fori_loop(0, nv, mbody, 0)
            pltpu.sync_copy(row_v, out_hbm.at[rid])
            return carry

        lax.fori_loop(0, rpw, per_row, jnp.int32(0))

    return topk_rows


def kernel(x):
    n, c, h, w = x.shape
    rows, row = n * c, h * w
    k = max(1, int(TOPK_FRAC * row))
    fn = _build(rows, row, k)
    out = fn(x.reshape(rows, row))
    return out.reshape(n, c, h, w)
